# baseline (speedup 1.0000x reference)
import jax
import jax.numpy as jnp
from jax import lax
from jax.experimental import pallas as pl
from jax.experimental.pallas import tpu as pltpu
from jax.experimental.pallas import tpu_sc as plsc

N = 10000
D = 128
E = 320000
C_OUT = 10

NC = 2
NS = 16
NW = NC * NS
EPW = E // NW
K = 80
CHUNKS = EPW // K
RPT = 624
REM = N - NS * RPT


def _sc_agg_body(h_hbm, eidx_hbm, out_hbm,
                 ring, rows0, rows1, rows2, rows3, acc,
                 isem0, isem1, isem2, isem3, isem4, isem5, isem6, isem7,
                 gsem0, gsem1, gsem2, gsem3, ssem0, ssem1, ssem2, ssem3,
                 hsem0, hsem1):
    c = lax.axis_index("c")
    s = lax.axis_index("s")
    wid = c * NS + s
    rows = (rows0, rows1, rows2, rows3)
    isems = (isem0, isem1, isem2, isem3, isem4, isem5, isem6, isem7)
    gsems = (gsem0, gsem1, gsem2, gsem3)
    ssems = (ssem0, ssem1, ssem2, ssem3)

    NZC = RPT // K
    ZT = RPT - NZC * K

    @pl.when(c == 0)
    def _():
        pltpu.async_copy(h_hbm.at[pl.ds(s * RPT, RPT)],
                         acc.at[pl.ds(s * RPT, RPT)], hsem0)

        @pl.when(s == 0)
        def _():
            pltpu.async_copy(h_hbm.at[pl.ds(NS * RPT, REM)],
                             acc.at[pl.ds(NS * RPT, REM)], hsem1)

    @pl.when(c == 1)
    def _():
        def _zrow(r, carry):
            for jj in range(D // 16):
                rows3[r, pl.ds(jj * 16, 16)] = jnp.zeros((16,), jnp.float32)
            return carry

        lax.fori_loop(0, K, _zrow, 0)
        for i in range(NZC):
            pltpu.async_copy(rows3, acc.at[pl.ds(s * RPT + i * K, K)], hsem0)
        pltpu.async_copy(rows3.at[pl.ds(0, ZT)],
                         acc.at[pl.ds(s * RPT + NZC * K, ZT)], hsem0)

        @pl.when(s == 0)
        def _():
            pltpu.async_copy(rows3.at[pl.ds(0, REM)],
                             acc.at[pl.ds(NS * RPT, REM)], hsem1)

    def _iload(g, t):
        pltpu.async_copy(eidx_hbm.at[wid, g], ring.at[t], isems[t])

    def _iwait(t):
        pltpu.make_async_copy(eidx_hbm.at[0, 0], ring.at[t],
                              isems[t]).wait()

    def _gather(t, b):
        pltpu.async_copy(h_hbm.at[ring.at[t, 0]], rows[b], gsems[b])

    def _gdrain(b):
        pltpu.make_async_copy(h_hbm.at[pl.ds(0, K)], rows[b],
                              gsems[b]).wait()

    def _scat(t, b):
        pltpu.async_copy(rows[b], acc.at[ring.at[t, 1]], ssems[b], add=True)

    def _sdrain(b):
        pltpu.make_async_copy(h_hbm.at[pl.ds(0, K)], rows[b],
                              ssems[b]).wait()

    for t in range(6):
        _iload(t, t)
    _iwait(0)
    _gather(0, 0)
    _iwait(1)
    _gather(1, 1)

    @pl.when(c == 0)
    def _():
        pltpu.make_async_copy(h_hbm.at[pl.ds(0, RPT)],
                              acc.at[pl.ds(s * RPT, RPT)], hsem0).wait()

    @pl.when(c == 1)
    def _():
        for i in range(NZC):
            pltpu.make_async_copy(h_hbm.at[pl.ds(0, K)], rows3, hsem0).wait()
        pltpu.make_async_copy(h_hbm.at[pl.ds(0, ZT)],
                              rows3.at[pl.ds(0, ZT)], hsem0).wait()

    @pl.when(s == 0)
    def _():
        pltpu.make_async_copy(h_hbm.at[pl.ds(0, REM)],
                              acc.at[pl.ds(NS * RPT, REM)], hsem1).wait()

    plsc.subcore_barrier()

    def _oct(q, carry):
        g = 8 * q
        for j in range(8):
            ch = g + j
            b, t = j % 4, j
            _gdrain(b)
            _scat(t, b)

            @pl.when(ch + 2 < CHUNKS)
            def _():
                @pl.when(ch >= 2)
                def _():
                    _sdrain((b + 2) % 4)
                _iwait((t + 2) % 8)
                _gather((t + 2) % 8, (b + 2) % 4)

            @pl.when(ch + 6 < CHUNKS)
            def _():
                _iload(ch + 6, (t + 6) % 8)

        return carry

    lax.fori_loop(0, CHUNKS // 8, _oct, 0)
    for ch in range((CHUNKS // 8) * 8, CHUNKS):
        b, t = ch % 4, ch % 8
        _gdrain(b)
        _scat(t, b)
        if ch + 2 < CHUNKS:
            _sdrain((b + 2) % 4)
            _iwait((t + 2) % 8)
            _gather((t + 2) % 8, (b + 2) % 4)
    for b in range(4):
        _sdrain(b)
    plsc.subcore_barrier()

    pltpu.sync_copy(acc.at[pl.ds(s * RPT, RPT)],
                    out_hbm.at[pl.ds(c * N + s * RPT, RPT)])

    @pl.when(s == 0)
    def _():
        pltpu.sync_copy(acc.at[pl.ds(NS * RPT, REM)],
                        out_hbm.at[pl.ds(c * N + NS * RPT, REM)])


_SC_AGG_CACHE = {}


def _sc_agg(h, eidx):
    if "k" not in _SC_AGG_CACHE:
        _SC_AGG_CACHE["k"] = pl.kernel(
            _sc_agg_body,
            out_type=jax.ShapeDtypeStruct((2 * N, D), jnp.float32),
            mesh=plsc.VectorSubcoreMesh(core_axis_name="c",
                                        subcore_axis_name="s"),
            scratch_types=[
                pltpu.VMEM((8, 2, K), jnp.int32),
                pltpu.VMEM((K, D), jnp.float32),
                pltpu.VMEM((K, D), jnp.float32),
                pltpu.VMEM((K, D), jnp.float32),
                pltpu.VMEM((K, D), jnp.float32),
                pltpu.VMEM_SHARED((N, D), jnp.float32),
            ] + [pltpu.SemaphoreType.DMA] * 18,
        )
    return _SC_AGG_CACHE["k"](h, eidx)


BLK = 2000
GRID = N // BLK

_row_spec = pl.BlockSpec((BLK, D), lambda i: (i, 0))
_pb_spec = pl.BlockSpec((BLK, D), lambda i: (i + GRID, 0))
_w_spec = pl.BlockSpec((D, D), lambda i: (0, 0))
_b_spec = pl.BlockSpec((1, D), lambda i: (0, 0))


def _mlp_body(pa_ref, pb_ref, w1_ref, b1_ref, w2_ref, b2_ref, o_ref):
    m = pa_ref[...] + pb_ref[...]
    t = jnp.maximum(
        jnp.dot(m, w1_ref[...], preferred_element_type=jnp.float32)
        + b1_ref[...], 0.0)
    o_ref[...] = jnp.maximum(
        jnp.dot(t, w2_ref[...], preferred_element_type=jnp.float32)
        + b2_ref[...], 0.0)


_mlp1 = pl.pallas_call(
    _mlp_body,
    grid=(GRID,),
    in_specs=[_row_spec, _pb_spec, _w_spec, _b_spec, _w_spec, _b_spec],
    out_specs=_row_spec,
    out_shape=jax.ShapeDtypeStruct((N, D), jnp.float32),
)


def _mlp_pool_body(pa_ref, pb_ref, w1_ref, b1_ref, w2_ref, b2_ref,
                   wo_ref, bo_ref, o_ref, acc_ref):
    i = pl.program_id(0)
    m = pa_ref[...] + pb_ref[...]
    t = jnp.maximum(
        jnp.dot(m, w1_ref[...], preferred_element_type=jnp.float32)
        + b1_ref[...], 0.0)
    h2 = jnp.maximum(
        jnp.dot(t, w2_ref[...], preferred_element_type=jnp.float32)
        + b2_ref[...], 0.0)
    ps = jnp.sum(h2, axis=0, keepdims=True)

    @pl.when(i == 0)
    def _():
        acc_ref[...] = ps

    @pl.when(i != 0)
    def _():
        acc_ref[...] = acc_ref[...] + ps

    @pl.when(i == GRID - 1)
    def _():
        o_ref[...] = (jnp.dot(acc_ref[...] * 2.0, wo_ref[...],
                              preferred_element_type=jnp.float32)
                      + bo_ref[...])


_mlp2 = pl.pallas_call(
    _mlp_pool_body,
    grid=(GRID,),
    in_specs=[_row_spec, _pb_spec, _w_spec, _b_spec, _w_spec, _b_spec,
              pl.BlockSpec((D, C_OUT), lambda i: (0, 0)),
              pl.BlockSpec((1, C_OUT), lambda i: (0, 0))],
    out_specs=pl.BlockSpec((1, C_OUT), lambda i: (0, 0)),
    out_shape=jax.ShapeDtypeStruct((1, C_OUT), jnp.float32),
    scratch_shapes=[pltpu.VMEM((1, D), jnp.float32)],
)


def kernel(x, edge_index, train_index, target_index, W1a, b1a, W2a, b2a,
           W1b, b1b, W2b, b2b, Wout, bout, att_train_k, att_target_k,
           att_train_q, att_target_q):
    eidx = (edge_index.astype(jnp.int32)
            .reshape(2, NW, CHUNKS, K).transpose(1, 2, 0, 3))
    p1 = _sc_agg(x, eidx)
    h1 = _mlp1(p1, p1, W1a, b1a.reshape(1, D), W2a, b2a.reshape(1, D))
    p2 = _sc_agg(h1, eidx)
    out = _mlp2(p2, p2, W1b, b1b.reshape(1, D), W2b, b2b.reshape(1, D),
                Wout, bout.reshape(1, C_OUT))
    return out


# speedup vs baseline: 1.0083x; 1.0083x over previous
import jax
import jax.numpy as jnp
from jax import lax
from jax.experimental import pallas as pl
from jax.experimental.pallas import tpu as pltpu
from jax.experimental.pallas import tpu_sc as plsc

N = 10000
D = 128
E = 320000
C_OUT = 10

NC = 2
NS = 16
NW = NC * NS
EPW = E // NW
K = 80
CHUNKS = EPW // K
RPT = 624
REM = N - NS * RPT


def _sc_agg_body(h_hbm, eidx_hbm, out_hbm,
                 ring, rows0, rows1, rows2, rows3, acc,
                 isem0, isem1, isem2, isem3, isem4, isem5, isem6, isem7,
                 gsem0, gsem1, gsem2, gsem3, ssem0, ssem1, ssem2, ssem3,
                 hsem0, hsem1):
    c = lax.axis_index("c")
    s = lax.axis_index("s")
    wid = c * NS + s
    rows = (rows0, rows1, rows2, rows3)
    isems = (isem0, isem1, isem2, isem3, isem4, isem5, isem6, isem7)
    gsems = (gsem0, gsem1, gsem2, gsem3)
    ssems = (ssem0, ssem1, ssem2, ssem3)

    NZC = RPT // K
    ZT = RPT - NZC * K

    @pl.when(c == 0)
    def _():
        pltpu.async_copy(h_hbm.at[pl.ds(s * RPT, RPT)],
                         acc.at[pl.ds(s * RPT, RPT)], hsem0)

        @pl.when(s == 0)
        def _():
            pltpu.async_copy(h_hbm.at[pl.ds(NS * RPT, REM)],
                             acc.at[pl.ds(NS * RPT, REM)], hsem1)

    @pl.when(c == 1)
    def _():
        def _zrow(r, carry):
            for jj in range(D // 16):
                rows3[r, pl.ds(jj * 16, 16)] = jnp.zeros((16,), jnp.float32)
            return carry

        lax.fori_loop(0, K, _zrow, 0)
        for i in range(NZC):
            pltpu.async_copy(rows3, acc.at[pl.ds(s * RPT + i * K, K)], hsem0)
        pltpu.async_copy(rows3.at[pl.ds(0, ZT)],
                         acc.at[pl.ds(s * RPT + NZC * K, ZT)], hsem0)

        @pl.when(s == 0)
        def _():
            pltpu.async_copy(rows3.at[pl.ds(0, REM)],
                             acc.at[pl.ds(NS * RPT, REM)], hsem1)

    def _iload(g, t):
        pltpu.async_copy(eidx_hbm.at[wid, g], ring.at[t], isems[t])

    def _iwait(t):
        pltpu.make_async_copy(eidx_hbm.at[0, 0], ring.at[t],
                              isems[t]).wait()

    def _gather(t, b):
        pltpu.async_copy(h_hbm.at[ring.at[t, 0]], rows[b], gsems[b])

    def _gdrain(b):
        pltpu.make_async_copy(h_hbm.at[pl.ds(0, K)], rows[b],
                              gsems[b]).wait()

    def _scat(t, b):
        pltpu.async_copy(rows[b], acc.at[ring.at[t, 1]], ssems[b], add=True)

    def _sdrain(b):
        pltpu.make_async_copy(h_hbm.at[pl.ds(0, K)], rows[b],
                              ssems[b]).wait()

    for t in range(6):
        _iload(t, t)
    _iwait(0)
    _gather(0, 0)
    _iwait(1)
    _gather(1, 1)
    _iwait(2)
    _gather(2, 2)

    @pl.when(c == 0)
    def _():
        pltpu.make_async_copy(h_hbm.at[pl.ds(0, RPT)],
                              acc.at[pl.ds(s * RPT, RPT)], hsem0).wait()

    @pl.when(c == 1)
    def _():
        for i in range(NZC):
            pltpu.make_async_copy(h_hbm.at[pl.ds(0, K)], rows3, hsem0).wait()
        pltpu.make_async_copy(h_hbm.at[pl.ds(0, ZT)],
                              rows3.at[pl.ds(0, ZT)], hsem0).wait()

    @pl.when(s == 0)
    def _():
        pltpu.make_async_copy(h_hbm.at[pl.ds(0, REM)],
                              acc.at[pl.ds(NS * RPT, REM)], hsem1).wait()

    plsc.subcore_barrier()

    def _oct(q, carry):
        g = 8 * q
        for j in range(8):
            ch = g + j
            b, t = j % 4, j
            _gdrain(b)
            _scat(t, b)

            @pl.when(ch + 2 < CHUNKS)
            def _():
                @pl.when(ch >= 2)
                def _():
                    _sdrain((b + 2) % 4)

                @pl.when(ch >= 1)
                def _():
                    _iwait((t + 2) % 8)
                    _gather((t + 2) % 8, (b + 2) % 4)

            @pl.when(ch + 6 < CHUNKS)
            def _():
                _iload(ch + 6, (t + 6) % 8)

        return carry

    lax.fori_loop(0, CHUNKS // 8, _oct, 0)
    for ch in range((CHUNKS // 8) * 8, CHUNKS):
        b, t = ch % 4, ch % 8
        _gdrain(b)
        _scat(t, b)
        if ch + 2 < CHUNKS:
            _sdrain((b + 2) % 4)
            _iwait((t + 2) % 8)
            _gather((t + 2) % 8, (b + 2) % 4)
    for b in range(4):
        _sdrain(b)
    plsc.subcore_barrier()

    pltpu.sync_copy(acc.at[pl.ds(s * RPT, RPT)],
                    out_hbm.at[pl.ds(c * N + s * RPT, RPT)])

    @pl.when(s == 0)
    def _():
        pltpu.sync_copy(acc.at[pl.ds(NS * RPT, REM)],
                        out_hbm.at[pl.ds(c * N + NS * RPT, REM)])


_SC_AGG_CACHE = {}


def _sc_agg(h, eidx):
    if "k" not in _SC_AGG_CACHE:
        _SC_AGG_CACHE["k"] = pl.kernel(
            _sc_agg_body,
            out_type=jax.ShapeDtypeStruct((2 * N, D), jnp.float32),
            mesh=plsc.VectorSubcoreMesh(core_axis_name="c",
                                        subcore_axis_name="s"),
            scratch_types=[
                pltpu.VMEM((8, 2, K), jnp.int32),
                pltpu.VMEM((K, D), jnp.float32),
                pltpu.VMEM((K, D), jnp.float32),
                pltpu.VMEM((K, D), jnp.float32),
                pltpu.VMEM((K, D), jnp.float32),
                pltpu.VMEM_SHARED((N, D), jnp.float32),
            ] + [pltpu.SemaphoreType.DMA] * 18,
        )
    return _SC_AGG_CACHE["k"](h, eidx)


BLK = 2000
GRID = N // BLK

_row_spec = pl.BlockSpec((BLK, D), lambda i: (i, 0))
_pb_spec = pl.BlockSpec((BLK, D), lambda i: (i + GRID, 0))
_w_spec = pl.BlockSpec((D, D), lambda i: (0, 0))
_b_spec = pl.BlockSpec((1, D), lambda i: (0, 0))


def _mlp_body(pa_ref, pb_ref, w1_ref, b1_ref, w2_ref, b2_ref, o_ref):
    m = pa_ref[...] + pb_ref[...]
    t = jnp.maximum(
        jnp.dot(m, w1_ref[...], preferred_element_type=jnp.float32)
        + b1_ref[...], 0.0)
    o_ref[...] = jnp.maximum(
        jnp.dot(t, w2_ref[...], preferred_element_type=jnp.float32)
        + b2_ref[...], 0.0)


_mlp1 = pl.pallas_call(
    _mlp_body,
    grid=(GRID,),
    in_specs=[_row_spec, _pb_spec, _w_spec, _b_spec, _w_spec, _b_spec],
    out_specs=_row_spec,
    out_shape=jax.ShapeDtypeStruct((N, D), jnp.float32),
)


def _mlp_pool_body(pa_ref, pb_ref, w1_ref, b1_ref, w2_ref, b2_ref,
                   wo_ref, bo_ref, o_ref, acc_ref):
    i = pl.program_id(0)
    m = pa_ref[...] + pb_ref[...]
    t = jnp.maximum(
        jnp.dot(m, w1_ref[...], preferred_element_type=jnp.float32)
        + b1_ref[...], 0.0)
    h2 = jnp.maximum(
        jnp.dot(t, w2_ref[...], preferred_element_type=jnp.float32)
        + b2_ref[...], 0.0)
    ps = jnp.sum(h2, axis=0, keepdims=True)

    @pl.when(i == 0)
    def _():
        acc_ref[...] = ps

    @pl.when(i != 0)
    def _():
        acc_ref[...] = acc_ref[...] + ps

    @pl.when(i == GRID - 1)
    def _():
        o_ref[...] = (jnp.dot(acc_ref[...] * 2.0, wo_ref[...],
                              preferred_element_type=jnp.float32)
                      + bo_ref[...])


_mlp2 = pl.pallas_call(
    _mlp_pool_body,
    grid=(GRID,),
    in_specs=[_row_spec, _pb_spec, _w_spec, _b_spec, _w_spec, _b_spec,
              pl.BlockSpec((D, C_OUT), lambda i: (0, 0)),
              pl.BlockSpec((1, C_OUT), lambda i: (0, 0))],
    out_specs=pl.BlockSpec((1, C_OUT), lambda i: (0, 0)),
    out_shape=jax.ShapeDtypeStruct((1, C_OUT), jnp.float32),
    scratch_shapes=[pltpu.VMEM((1, D), jnp.float32)],
)


def kernel(x, edge_index, train_index, target_index, W1a, b1a, W2a, b2a,
           W1b, b1b, W2b, b2b, Wout, bout, att_train_k, att_target_k,
           att_train_q, att_target_q):
    eidx = (edge_index.astype(jnp.int32)
            .reshape(2, NW, CHUNKS, K).transpose(1, 2, 0, 3))
    p1 = _sc_agg(x, eidx)
    h1 = _mlp1(p1, p1, W1a, b1a.reshape(1, D), W2a, b2a.reshape(1, D))
    p2 = _sc_agg(h1, eidx)
    out = _mlp2(p2, p2, W1b, b1b.reshape(1, D), W2b, b2b.reshape(1, D),
                Wout, bout.reshape(1, C_OUT))
    return out
